# Initial kernel scaffold; baseline (speedup 1.0000x reference)
import jax
import jax.numpy as jnp
from jax.experimental import pallas as pl


def kernel(x, weight):
    raise NotImplementedError("write your pallas kernel here")


# speedup vs baseline: 1.0938x; 1.0938x over previous
import functools

import jax
import jax.numpy as jnp
from jax import lax
from jax.experimental import pallas as pl
from jax.experimental.pallas import tpu as pltpu
from jax.experimental.pallas import tpu_sc as plsc

NUM_ROWS = 1000000
DIM = 32
BATCH = 16384 * 50

NUM_CORES = 2
NUM_SUBCORES = 16
NUM_WORKERS = NUM_CORES * NUM_SUBCORES
ROWS_PER_WORKER = BATCH // NUM_WORKERS
CHUNK = 1024
NUM_CHUNKS = ROWS_PER_WORKER // CHUNK

_mesh = plsc.VectorSubcoreMesh(core_axis_name="c", subcore_axis_name="s")


@functools.partial(
    pl.kernel,
    mesh=_mesh,
    out_type=jax.ShapeDtypeStruct((BATCH, DIM), jnp.float32),
    scratch_types=[
        pltpu.VMEM((CHUNK,), jnp.int32),
        pltpu.VMEM((CHUNK, DIM), jnp.float32),
        pltpu.SemaphoreType.DMA,
    ],
    compiler_params=pltpu.CompilerParams(use_tc_tiling_on_sc=False),
)
def _gather_kernel(idx_hbm, table_hbm, out_hbm, idx_v, rows_v, sem):
    wid = lax.axis_index("s") * NUM_CORES + lax.axis_index("c")
    base = wid * ROWS_PER_WORKER

    def body(g, carry):
        off = base + g * CHUNK
        pltpu.sync_copy(idx_hbm.at[pl.ds(off, CHUNK)], idx_v)
        pltpu.async_copy(table_hbm.at[idx_v], rows_v, sem).wait()
        pltpu.sync_copy(rows_v, out_hbm.at[pl.ds(off, CHUNK)])
        return carry

    lax.fori_loop(0, NUM_CHUNKS, body, 0)


def kernel(x, weight):
    idx = x.reshape(-1).astype(jnp.int32)
    out = _gather_kernel(idx, weight)
    return out.reshape(x.shape + (weight.shape[1],))


# speedup vs baseline: 1.1126x; 1.0173x over previous
import functools

import jax
import jax.numpy as jnp
from jax import lax
from jax.experimental import pallas as pl
from jax.experimental.pallas import tpu as pltpu
from jax.experimental.pallas import tpu_sc as plsc

NUM_ROWS = 1000000
DIM = 32
BATCH = 16384 * 50

NUM_CORES = 2
NUM_SUBCORES = 16
NUM_WORKERS = NUM_CORES * NUM_SUBCORES
ROWS_PER_WORKER = BATCH // NUM_WORKERS
CHUNK = 1600
NUM_CHUNKS = ROWS_PER_WORKER // CHUNK

_mesh = plsc.VectorSubcoreMesh(core_axis_name="c", subcore_axis_name="s")


@functools.partial(
    pl.kernel,
    mesh=_mesh,
    out_type=jax.ShapeDtypeStruct((BATCH, DIM), jnp.float32),
    scratch_types=[
        pltpu.VMEM((ROWS_PER_WORKER,), jnp.int32),
        pltpu.VMEM((CHUNK, DIM), jnp.float32),
        pltpu.VMEM((CHUNK, DIM), jnp.float32),
        pltpu.SemaphoreType.DMA,
        pltpu.SemaphoreType.DMA,
        pltpu.SemaphoreType.DMA,
        pltpu.SemaphoreType.DMA,
    ],
    compiler_params=pltpu.CompilerParams(use_tc_tiling_on_sc=False),
)
def _gather_kernel(idx_hbm, table_hbm, out_hbm, idx_v, rows0, rows1,
                   gsem0, gsem1, osem0, osem1):
    wid = lax.axis_index("s") * NUM_CORES + lax.axis_index("c")
    base = wid * ROWS_PER_WORKER

    rows = (rows0, rows1)
    gsem = (gsem0, gsem1)
    osem = (osem0, osem1)

    pltpu.sync_copy(idx_hbm.at[pl.ds(base, ROWS_PER_WORKER)], idx_v)

    def gather(g, p):
        pltpu.make_async_copy(
            table_hbm.at[idx_v.at[pl.ds(g * CHUNK, CHUNK)]],
            rows[p], gsem[p]).start()

    def writeback(g, p):
        pltpu.make_async_copy(
            rows[p], out_hbm.at[pl.ds(base + g * CHUNK, CHUNK)],
            osem[p]).start()

    gather(0, 0)
    for g in range(NUM_CHUNKS):
        p = g % 2
        q = 1 - p
        if g + 1 < NUM_CHUNKS:
            if g >= 1:
                pltpu.make_async_copy(
                    rows[q], out_hbm.at[pl.ds(base + (g - 1) * CHUNK, CHUNK)],
                    osem[q]).wait()
            gather(g + 1, q)
        pltpu.make_async_copy(
            table_hbm.at[idx_v.at[pl.ds(g * CHUNK, CHUNK)]],
            rows[p], gsem[p]).wait()
        writeback(g, p)

    last = NUM_CHUNKS - 1
    pltpu.make_async_copy(
        rows[(last - 1) % 2],
        out_hbm.at[pl.ds(base + (last - 1) * CHUNK, CHUNK)],
        osem[(last - 1) % 2]).wait()
    pltpu.make_async_copy(
        rows[last % 2], out_hbm.at[pl.ds(base + last * CHUNK, CHUNK)],
        osem[last % 2]).wait()


def kernel(x, weight):
    idx = x.reshape(-1).astype(jnp.int32)
    out = _gather_kernel(idx, weight)
    return out.reshape(x.shape + (weight.shape[1],))


# speedup vs baseline: 1.4754x; 1.3261x over previous
import functools

import jax
import jax.numpy as jnp
from jax import lax
from jax.experimental import pallas as pl
from jax.experimental.pallas import tpu as pltpu
from jax.experimental.pallas import tpu_sc as plsc

NUM_ROWS = 1000000
DIM = 32
NSAMP = 16384
NSLOT = 50
NROW4 = NUM_ROWS // 4

NUM_CORES = 2
NUM_SUBCORES = 16
NUM_WORKERS = NUM_CORES * NUM_SUBCORES
SBLK_PER_W = (NSAMP // 128) // NUM_WORKERS

_mesh = plsc.VectorSubcoreMesh(core_axis_name="c", subcore_axis_name="s")


@functools.partial(
    pl.kernel,
    mesh=_mesh,
    out_type=jax.ShapeDtypeStruct((NSLOT, DIM, NSAMP), jnp.float32),
    scratch_types=[
        pltpu.VMEM((128 * NSLOT,), jnp.int32),
        pltpu.VMEM((128,), jnp.int32),
        pltpu.VMEM((128,), jnp.int32),
        pltpu.VMEM((128,), jnp.int32),
        pltpu.VMEM((128,), jnp.int32),
        pltpu.VMEM((128, 128), jnp.float32),
        pltpu.VMEM((128, 128), jnp.float32),
        pltpu.VMEM((DIM, 128), jnp.float32),
        pltpu.VMEM((DIM, 128), jnp.float32),
        pltpu.SemaphoreType.DMA,
        pltpu.SemaphoreType.DMA,
        pltpu.SemaphoreType.DMA,
        pltpu.SemaphoreType.DMA,
        pltpu.SemaphoreType.DMA,
    ],
    compiler_params=pltpu.CompilerParams(
        use_tc_tiling_on_sc=False, needs_layout_passes=False),
)
def _lookup_kernel(xf_hbm, t4_hbm, out_hbm, xchunk, idx0, idx1, qsh0, qsh1,
                   g0, g1, o0, o1, xsem, gsem0, gsem1, osem0, osem1):
    w = lax.axis_index("s") * NUM_CORES + lax.axis_index("c")
    idxs = (idx0, idx1)
    qshs = (qsh0, qsh1)
    gbufs = (g0, g1)
    obufs = (o0, o1)
    gsems = (gsem0, gsem1)
    osems = (osem0, osem1)
    iota = jax.lax.iota(jnp.int32, 16)

    def prep_and_fire(l, p):
        for m in range(8):
            v = plsc.load_gather(xchunk, [(iota + m * 16) * NSLOT + l])
            idxs[p][pl.ds(m * 16, 16)] = v >> 2
            qshs[p][pl.ds(m * 16, 16)] = (v & 3) << 5
        pltpu.make_async_copy(t4_hbm.at[idxs[p]], gbufs[p], gsems[p]).start()

    def wait_g(p):
        pltpu.make_async_copy(t4_hbm.at[idxs[p]], gbufs[p], gsems[p]).wait()

    def transpose_sel(p):
        qvs = tuple(qshs[p][pl.ds(m * 16, 16)] for m in range(8))

        def body(d, qvs):
            dsplat = jnp.full((16,), d, jnp.int32)
            for m in range(8):
                v = plsc.load_gather(gbufs[p], [iota + m * 16, qvs[m] + d])
                plsc.store_scatter(obufs[p], [dsplat, iota + m * 16], v)
            return qvs
        lax.fori_loop(0, DIM, body, qvs)

    def start_out(sb, l, p):
        pltpu.make_async_copy(
            obufs[p], out_hbm.at[l, :, pl.ds(sb * 128, 128)], osems[p]).start()

    def wait_out(p):
        pltpu.make_async_copy(
            obufs[p], out_hbm.at[0, :, pl.ds(0, 128)], osems[p]).wait()

    for ss in range(SBLK_PER_W):
        sb = w * SBLK_PER_W + ss
        pltpu.sync_copy(xf_hbm.at[pl.ds(sb * 128 * NSLOT, 128 * NSLOT)],
                        xchunk)
        prep_and_fire(0, 0)
        prep_and_fire(1, 1)

        def half(m, p, l):
            wait_g(p)

            @pl.when(m > 0)
            def _():
                wait_out(p)

            transpose_sel(p)
            start_out(sb, l, p)

            @pl.when(m < NSLOT // 2 - 1)
            def _():
                prep_and_fire(l + 2, p)

        def body(m, carry):
            half(m, 0, 2 * m)
            half(m, 1, 2 * m + 1)
            return carry

        lax.fori_loop(0, NSLOT // 2, body, 0)
        wait_out(0)
        wait_out(1)


def kernel(x, weight):
    xf = x.reshape(-1).astype(jnp.int32)
    t4 = weight.reshape(NROW4, 128)
    outp = _lookup_kernel(xf, t4)
    return outp.transpose(2, 0, 1)


# speedup vs baseline: 1.4844x; 1.0061x over previous
import functools

import jax
import jax.numpy as jnp
from jax import lax
from jax.experimental import pallas as pl
from jax.experimental.pallas import tpu as pltpu
from jax.experimental.pallas import tpu_sc as plsc

NUM_ROWS = 1000000
DIM = 32
NSAMP = 16384
NSLOT = 50

NUM_CORES = 2
NUM_SUBCORES = 16
NUM_WORKERS = NUM_CORES * NUM_SUBCORES
SBLK_PER_W = (NSAMP // 128) // NUM_WORKERS

_mesh = plsc.VectorSubcoreMesh(core_axis_name="c", subcore_axis_name="s")


@functools.partial(
    pl.kernel,
    mesh=_mesh,
    out_type=jax.ShapeDtypeStruct((NSLOT, DIM, NSAMP), jnp.float32),
    scratch_types=[
        pltpu.VMEM((128 * NSLOT,), jnp.int32),
        pltpu.VMEM((128,), jnp.int32),
        pltpu.VMEM((128,), jnp.int32),
        pltpu.VMEM((128, DIM), jnp.float32),
        pltpu.VMEM((128, DIM), jnp.float32),
        pltpu.VMEM((DIM, 128), jnp.float32),
        pltpu.VMEM((DIM, 128), jnp.float32),
        pltpu.SemaphoreType.DMA,
        pltpu.SemaphoreType.DMA,
        pltpu.SemaphoreType.DMA,
        pltpu.SemaphoreType.DMA,
        pltpu.SemaphoreType.DMA,
    ],
    compiler_params=pltpu.CompilerParams(
        use_tc_tiling_on_sc=False, needs_layout_passes=False),
)
def _lookup_kernel(xf_hbm, w_hbm, out_hbm, xchunk, idx0, idx1,
                   g0, g1, o0, o1, xsem, gsem0, gsem1, osem0, osem1):
    w = lax.axis_index("s") * NUM_CORES + lax.axis_index("c")
    idxs = (idx0, idx1)
    gbufs = (g0, g1)
    obufs = (o0, o1)
    gsems = (gsem0, gsem1)
    osems = (osem0, osem1)
    iota = jax.lax.iota(jnp.int32, 16)

    def prep_and_fire(l, p):
        for m in range(8):
            v = plsc.load_gather(xchunk, [(iota + m * 16) * NSLOT + l])
            idxs[p][pl.ds(m * 16, 16)] = v
        pltpu.make_async_copy(w_hbm.at[idxs[p]], gbufs[p], gsems[p]).start()

    def wait_g(p):
        pltpu.make_async_copy(w_hbm.at[idxs[p]], gbufs[p], gsems[p]).wait()

    def transpose_sel(p):
        def body(d, carry):
            dsplat = jnp.full((16,), d, jnp.int32)
            for m in range(8):
                v = plsc.load_gather(gbufs[p], [iota + m * 16, dsplat])
                plsc.store_scatter(obufs[p], [dsplat, iota + m * 16], v)
            return carry
        lax.fori_loop(0, DIM, body, 0)

    def start_out(sb, l, p):
        pltpu.make_async_copy(
            obufs[p], out_hbm.at[l, :, pl.ds(sb * 128, 128)], osems[p]).start()

    def wait_out(p):
        pltpu.make_async_copy(
            obufs[p], out_hbm.at[0, :, pl.ds(0, 128)], osems[p]).wait()

    for ss in range(SBLK_PER_W):
        sb = w * SBLK_PER_W + ss
        pltpu.sync_copy(xf_hbm.at[pl.ds(sb * 128 * NSLOT, 128 * NSLOT)],
                        xchunk)
        prep_and_fire(0, 0)
        prep_and_fire(1, 1)

        def half(m, p, l):
            wait_g(p)

            @pl.when(m > 0)
            def _():
                wait_out(p)

            transpose_sel(p)
            start_out(sb, l, p)

            @pl.when(m < NSLOT // 2 - 1)
            def _():
                prep_and_fire(l + 2, p)

        def body(m, carry):
            half(m, 0, 2 * m)
            half(m, 1, 2 * m + 1)
            return carry

        lax.fori_loop(0, NSLOT // 2, body, 0)
        wait_out(0)
        wait_out(1)


def kernel(x, weight):
    xf = x.reshape(-1).astype(jnp.int32)
    outp = _lookup_kernel(xf, weight)
    return outp.transpose(2, 0, 1)


# speedup vs baseline: 2.1955x; 1.4791x over previous
import functools

import jax
import jax.numpy as jnp
from jax import lax
from jax.experimental import pallas as pl
from jax.experimental.pallas import tpu as pltpu
from jax.experimental.pallas import tpu_sc as plsc

NUM_ROWS = 1000000
DIM = 32
NSAMP = 16384
NSLOT = 50

NUM_CORES = 2
NUM_SUBCORES = 16
NUM_WORKERS = NUM_CORES * NUM_SUBCORES
SBLK_PER_W = (NSAMP // 128) // NUM_WORKERS

_mesh = plsc.VectorSubcoreMesh(core_axis_name="c", subcore_axis_name="s")


@functools.partial(
    pl.kernel,
    mesh=_mesh,
    out_type=jax.ShapeDtypeStruct((NSLOT, DIM, NSAMP), jnp.float32),
    scratch_types=[
        pltpu.VMEM((128 * NSLOT,), jnp.int32),
        pltpu.VMEM((128,), jnp.int32),
        pltpu.VMEM((128,), jnp.int32),
        pltpu.VMEM((128, DIM), jnp.float32),
        pltpu.VMEM((128, DIM), jnp.float32),
        pltpu.VMEM((DIM, 128), jnp.float32),
        pltpu.VMEM((DIM, 128), jnp.float32),
        pltpu.SemaphoreType.DMA,
        pltpu.SemaphoreType.DMA,
        pltpu.SemaphoreType.DMA,
        pltpu.SemaphoreType.DMA,
        pltpu.SemaphoreType.DMA,
    ],
    compiler_params=pltpu.CompilerParams(
        use_tc_tiling_on_sc=False, needs_layout_passes=False),
)
def _lookup_kernel(xf_hbm, w_hbm, out_hbm, xchunk, idx0, idx1,
                   g0, g1, o0, o1, xsem, gsem0, gsem1, osem0, osem1):
    w = lax.axis_index("s") * NUM_CORES + lax.axis_index("c")
    idxs = (idx0, idx1)
    gbufs = (g0, g1)
    obufs = (o0, o1)
    gsems = (gsem0, gsem1)
    osems = (osem0, osem1)
    iota = jax.lax.iota(jnp.int32, 16)

    def prep_and_fire(l, p):
        for m in range(8):
            v = plsc.load_gather(xchunk, [(iota + m * 16) * NSLOT + l])
            idxs[p][pl.ds(m * 16, 16)] = v
        pltpu.make_async_copy(w_hbm.at[idxs[p]], gbufs[p], gsems[p]).start()

    def wait_g(p):
        pltpu.make_async_copy(w_hbm.at[idxs[p]], gbufs[p], gsems[p]).wait()

    def transpose_sel(p):
        def body(d0, carry):
            dv = (d0 + iota) & (DIM - 1)
            for m in range(8):
                v = plsc.load_gather(gbufs[p], [iota + m * 16, dv])
                plsc.store_scatter(obufs[p], [dv, iota + m * 16], v)
            return carry
        lax.fori_loop(0, DIM, body, 0)

    def start_out(sb, l, p):
        pltpu.make_async_copy(
            obufs[p], out_hbm.at[l, :, pl.ds(sb * 128, 128)], osems[p]).start()

    def wait_out(p):
        pltpu.make_async_copy(
            obufs[p], out_hbm.at[0, :, pl.ds(0, 128)], osems[p]).wait()

    for ss in range(SBLK_PER_W):
        sb = w * SBLK_PER_W + ss
        pltpu.sync_copy(xf_hbm.at[pl.ds(sb * 128 * NSLOT, 128 * NSLOT)],
                        xchunk)
        prep_and_fire(0, 0)
        prep_and_fire(1, 1)

        def half(m, p, l):
            wait_g(p)

            @pl.when(m > 0)
            def _():
                wait_out(p)

            transpose_sel(p)
            start_out(sb, l, p)

            @pl.when(m < NSLOT // 2 - 1)
            def _():
                prep_and_fire(l + 2, p)

        def body(m, carry):
            half(m, 0, 2 * m)
            half(m, 1, 2 * m + 1)
            return carry

        lax.fori_loop(0, NSLOT // 2, body, 0)
        wait_out(0)
        wait_out(1)


def kernel(x, weight):
    xf = x.reshape(-1).astype(jnp.int32)
    outp = _lookup_kernel(xf, weight)
    return outp.transpose(2, 0, 1)


# speedup vs baseline: 2.5299x; 1.1523x over previous
import functools

import jax
import jax.numpy as jnp
from jax import lax
from jax.experimental import pallas as pl
from jax.experimental.pallas import tpu as pltpu
from jax.experimental.pallas import tpu_sc as plsc

NUM_ROWS = 1000000
DIM = 32
NSAMP = 16384
NSLOT = 50

NUM_CORES = 2
NUM_SUBCORES = 16
NUM_WORKERS = NUM_CORES * NUM_SUBCORES
SBLK_PER_W = (NSAMP // 128) // NUM_WORKERS

_mesh = plsc.VectorSubcoreMesh(core_axis_name="c", subcore_axis_name="s")


@functools.partial(
    pl.kernel,
    mesh=_mesh,
    out_type=jax.ShapeDtypeStruct((NSLOT, 4, NSAMP // 128, 1024),
                                  jnp.float32),
    scratch_types=[
        pltpu.VMEM((128 * NSLOT,), jnp.int32),
        pltpu.VMEM((128,), jnp.int32),
        pltpu.VMEM((128,), jnp.int32),
        pltpu.VMEM((128, DIM), jnp.float32),
        pltpu.VMEM((128, DIM), jnp.float32),
        pltpu.VMEM((4, 1024), jnp.float32),
        pltpu.VMEM((4, 1024), jnp.float32),
        pltpu.SemaphoreType.DMA,
        pltpu.SemaphoreType.DMA,
        pltpu.SemaphoreType.DMA,
        pltpu.SemaphoreType.DMA,
        pltpu.SemaphoreType.DMA,
    ],
    compiler_params=pltpu.CompilerParams(
        use_tc_tiling_on_sc=False, needs_layout_passes=False),
)
def _lookup_kernel(xf_hbm, w_hbm, out_hbm, xchunk, idx0, idx1,
                   g0, g1, o0, o1, xsem, gsem0, gsem1, osem0, osem1):
    w = lax.axis_index("s") * NUM_CORES + lax.axis_index("c")
    idxs = (idx0, idx1)
    gbufs = (g0, g1)
    obufs = (o0, o1)
    gsems = (gsem0, gsem1)
    osems = (osem0, osem1)
    iota = jax.lax.iota(jnp.int32, 16)

    def prep_and_fire(l, p):
        for m in range(8):
            v = plsc.load_gather(xchunk, [(iota + m * 16) * NSLOT + l])
            idxs[p][pl.ds(m * 16, 16)] = v
        pltpu.make_async_copy(w_hbm.at[idxs[p]], gbufs[p], gsems[p]).start()

    def wait_g(p):
        pltpu.make_async_copy(w_hbm.at[idxs[p]], gbufs[p], gsems[p]).wait()

    def transpose_sel(p):
        def body(d0, carry):
            dv = (d0 + iota) & (DIM - 1)
            rv = dv >> 3
            cb = (dv & 7) << 7
            for m in range(8):
                v = plsc.load_gather(gbufs[p], [iota + m * 16, dv])
                plsc.store_scatter(obufs[p], [rv, cb + iota + m * 16], v)
            return carry
        lax.fori_loop(0, DIM, body, 0)

    def start_out(sb, l, p):
        pltpu.make_async_copy(
            obufs[p], out_hbm.at[l, :, sb, :], osems[p]).start()

    def wait_out(p):
        pltpu.make_async_copy(
            obufs[p], out_hbm.at[0, :, 0, :], osems[p]).wait()

    for ss in range(SBLK_PER_W):
        sb = w * SBLK_PER_W + ss
        pltpu.sync_copy(xf_hbm.at[pl.ds(sb * 128 * NSLOT, 128 * NSLOT)],
                        xchunk)
        prep_and_fire(0, 0)
        prep_and_fire(1, 1)

        def half(m, p, l):
            wait_g(p)

            @pl.when(m > 0)
            def _():
                wait_out(p)

            transpose_sel(p)
            start_out(sb, l, p)

            @pl.when(m < NSLOT // 2 - 1)
            def _():
                prep_and_fire(l + 2, p)

        def body(m, carry):
            half(m, 0, 2 * m)
            half(m, 1, 2 * m + 1)
            return carry

        lax.fori_loop(0, NSLOT // 2, body, 0)
        wait_out(0)
        wait_out(1)


def kernel(x, weight):
    xf = x.reshape(-1).astype(jnp.int32)
    outp = _lookup_kernel(xf, weight)
    out5 = outp.reshape(NSLOT, 4, NSAMP // 128, 8, 128)
    return out5.transpose(2, 4, 0, 1, 3).reshape(NSAMP, NSLOT, DIM)
